# baseline (speedup 1.0000x reference)
import functools

import jax
import jax.numpy as jnp
from jax.experimental import pallas as pl
from jax.experimental.pallas import tpu as pltpu

EPS = 1e-5
VMEM_LIMIT_BYTES = 48 * 1024 * 1024


def _conv_stats_kernel(x0_ref, x1_ref, x2_ref, x3_ref, w1_ref, y_ref,
                       stats_ref, xp_ref, s_ref, *, w, hw, pad, tile):
  cq = x0_ref.shape[0]
  cin = 4 * cq
  cmid = w1_ref.shape[1]
  xp_ref[:, :pad] = jnp.zeros((cin, pad), jnp.bfloat16)
  xp_ref[:, pad + hw:] = jnp.zeros((cin, pad), jnp.bfloat16)
  for q, xq_ref in enumerate((x0_ref, x1_ref, x2_ref, x3_ref)):
    xp_ref[q * cq:(q + 1) * cq, pad:pad + hw] = (
        xq_ref[...].astype(jnp.bfloat16).reshape(cq, hw))

  lane_w = jax.lax.broadcasted_iota(jnp.int32, (1, tile), 1) % w
  mask_l = (lane_w > 0).astype(jnp.float32)
  mask_r = (lane_w < w - 1).astype(jnp.float32)

  s = jnp.zeros((cmid, 1), jnp.float32)
  s2 = jnp.zeros((cmid, 1), jnp.float32)
  for t in range(hw // tile):
    base = pad + t * tile
    for dy in range(3):
      s_ref[dy * cin:(dy + 1) * cin, :] = (
          xp_ref[:, base + (dy - 1) * w:base + (dy - 1) * w + tile])
    p0 = jnp.dot(w1_ref[0], s_ref[...],
                 preferred_element_type=jnp.float32)
    p1 = jnp.dot(w1_ref[1], s_ref[...],
                 preferred_element_type=jnp.float32)
    p2 = jnp.dot(w1_ref[2], s_ref[...],
                 preferred_element_type=jnp.float32)
    acc = (p1 + mask_l * jnp.roll(p0, 1, axis=1)
           + mask_r * jnp.roll(p2, -1, axis=1))
    y_ref[:, t * tile:(t + 1) * tile] = acc.astype(y_ref.dtype)
    s = s + jnp.sum(acc, axis=1, keepdims=True)
    s2 = s2 + jnp.sum(acc * acc, axis=1, keepdims=True)
  stats_ref[:, 0:1] = s
  stats_ref[:, 1:2] = s2


def _apply_kernel(y_ref, scale_ref, shift_ref, w2_ref, b2_ref, o_ref):
  cout, r, w = o_ref.shape
  y = y_ref[...].astype(jnp.float32)
  h = jnp.maximum(y * scale_ref[...] + shift_ref[...], 0.0)
  z = jnp.dot(w2_ref[...], h, preferred_element_type=jnp.float32) + b2_ref[...]
  o_ref[...] = jax.nn.sigmoid(z).astype(o_ref.dtype).reshape(cout, r, w)


def kernel(x_nchw, w1_oihw, b1, gamma, beta, w2_oihw, b2):
  del b1
  N, Cin, H, W = x_nchw.shape
  Cmid = w1_oihw.shape[0]
  Cout = w2_oihw.shape[0]
  HW = H * W
  PAD = 2 * W
  TILE = min(8192, HW)
  count = float(N * H * W)

  w1g = jnp.transpose(w1_oihw, (3, 0, 2, 1)).reshape(3, Cmid, 3 * Cin)
  w1g = w1g.astype(jnp.bfloat16)
  w2m = w2_oihw[:, :, 0, 0]

  NC = 2 if N % 2 == 0 else 1
  NPC = N // NC
  cparams = pltpu.CompilerParams(
      dimension_semantics=("parallel", "arbitrary"),
      vmem_limit_bytes=VMEM_LIMIT_BYTES)

  y, stats = pl.pallas_call(
      functools.partial(_conv_stats_kernel, w=W, hw=HW, pad=PAD, tile=TILE),
      out_shape=(jax.ShapeDtypeStruct((N, Cmid, HW), jnp.bfloat16),
                 jax.ShapeDtypeStruct((N, Cmid, 2), jnp.float32)),
      grid=(NC, NPC),
      in_specs=[
          pl.BlockSpec((None, Cin // 4, H, W),
                       functools.partial(
                           lambda q, c, i: (c * NPC + i, q, 0, 0), q))
          for q in range(4)
      ] + [
          pl.BlockSpec((3, Cmid, 3 * Cin), lambda c, i: (0, 0, 0)),
      ],
      out_specs=(pl.BlockSpec((None, Cmid, HW),
                              lambda c, i: (c * NPC + i, 0, 0)),
                 pl.BlockSpec((None, Cmid, 2),
                              lambda c, i: (c * NPC + i, 0, 0))),
      scratch_shapes=[pltpu.VMEM((Cin, PAD + HW + PAD), jnp.bfloat16),
                      pltpu.VMEM((3 * Cin, TILE), jnp.bfloat16)],
      compiler_params=cparams,
      cost_estimate=pl.CostEstimate(
          flops=int((2 * 9 * Cin * Cmid + 4 * Cmid) * N * HW),
          transcendentals=0,
          bytes_accessed=int(4 * N * Cin * HW + 2 * N * Cmid * HW)),
  )(x_nchw, x_nchw, x_nchw, x_nchw, w1g)

  s = jnp.sum(stats[:, :, 0], axis=0)
  s2 = jnp.sum(stats[:, :, 1], axis=0)
  mean = s / count
  var = s2 / count - mean * mean
  scale = gamma * jax.lax.rsqrt(var + EPS)
  shift = beta - mean * scale

  T2 = min(16384, HW)
  R2 = T2 // W
  NT2 = HW // T2
  cparams2 = pltpu.CompilerParams(
      dimension_semantics=("parallel", "arbitrary"),
      vmem_limit_bytes=VMEM_LIMIT_BYTES)
  out = pl.pallas_call(
      _apply_kernel,
      out_shape=jax.ShapeDtypeStruct((N, Cout, H, W), jnp.float32),
      grid=(NC, NPC * NT2),
      in_specs=[
          pl.BlockSpec((None, Cmid, T2),
                       lambda c, j: (c * NPC + j // NT2, 0, j % NT2)),
          pl.BlockSpec((Cmid, 1), lambda c, j: (0, 0)),
          pl.BlockSpec((Cmid, 1), lambda c, j: (0, 0)),
          pl.BlockSpec((Cout, Cmid), lambda c, j: (0, 0)),
          pl.BlockSpec((Cout, 1), lambda c, j: (0, 0)),
      ],
      out_specs=pl.BlockSpec(
          (None, Cout, R2, W),
          lambda c, j: (c * NPC + j // NT2, 0, j % NT2, 0)),
      compiler_params=cparams2,
      cost_estimate=pl.CostEstimate(
          flops=int((2 * Cmid * Cout + 2 * Cmid + 2 * Cout) * N * HW),
          transcendentals=int(Cout * N * HW),
          bytes_accessed=int(4 * (N * Cmid * HW + N * Cout * HW))),
  )(y, scale.reshape(Cmid, 1), shift.reshape(Cmid, 1), w2m,
    b2.reshape(Cout, 1))

  return out


# speedup vs baseline: 1.0146x; 1.0146x over previous
import functools

import jax
import jax.numpy as jnp
from jax.experimental import pallas as pl
from jax.experimental.pallas import tpu as pltpu

EPS = 1e-5
VMEM_LIMIT_BYTES = 48 * 1024 * 1024


def _conv_stats_kernel(x0_ref, x1_ref, x2_ref, x3_ref, w1_ref, y_ref,
                       stats_ref, xp_ref, s_ref, *, w, hw, pad, tile):
  cq = x0_ref.shape[0]
  cin = 4 * cq
  cmid = w1_ref.shape[1]
  xp_ref[:, :pad] = jnp.zeros((cin, pad), jnp.bfloat16)
  xp_ref[:, pad + hw:] = jnp.zeros((cin, pad), jnp.bfloat16)
  for q, xq_ref in enumerate((x0_ref, x1_ref, x2_ref, x3_ref)):
    xp_ref[q * cq:(q + 1) * cq, pad:pad + hw] = (
        xq_ref[...].astype(jnp.bfloat16).reshape(cq, hw))

  lane_w = jax.lax.broadcasted_iota(jnp.int32, (1, tile), 1) % w
  mask_l = (lane_w > 0).astype(jnp.float32)
  mask_r = (lane_w < w - 1).astype(jnp.float32)

  s = jnp.zeros((cmid, 1), jnp.float32)
  s2 = jnp.zeros((cmid, 1), jnp.float32)
  for t in range(hw // tile):
    base = pad + t * tile
    for dy in range(3):
      s_ref[dy * cin:(dy + 1) * cin, :] = (
          xp_ref[:, base + (dy - 1) * w:base + (dy - 1) * w + tile])
    p0 = jnp.dot(w1_ref[0], s_ref[...],
                 preferred_element_type=jnp.float32)
    p1 = jnp.dot(w1_ref[1], s_ref[...],
                 preferred_element_type=jnp.float32)
    p2 = jnp.dot(w1_ref[2], s_ref[...],
                 preferred_element_type=jnp.float32)
    acc = (p1 + mask_l * jnp.roll(p0, 1, axis=1)
           + mask_r * jnp.roll(p2, -1, axis=1))
    y_ref[:, t * tile:(t + 1) * tile] = acc.astype(y_ref.dtype)
    s = s + jnp.sum(acc, axis=1, keepdims=True)
    s2 = s2 + jnp.sum(acc * acc, axis=1, keepdims=True)
  stats_ref[:, 0:1] = s
  stats_ref[:, 1:2] = s2


def _apply_kernel(y_ref, scale_ref, shift_ref, w2_ref, b2_ref, o_ref):
  cout, r, w = o_ref.shape
  y = y_ref[...].astype(jnp.float32)
  h = jnp.maximum(y * scale_ref[...] + shift_ref[...], 0.0)
  z = jnp.dot(w2_ref[...], h, preferred_element_type=jnp.float32) + b2_ref[...]
  o_ref[...] = jax.nn.sigmoid(z).astype(o_ref.dtype).reshape(cout, r, w)


def kernel(x_nchw, w1_oihw, b1, gamma, beta, w2_oihw, b2):
  del b1
  N, Cin, H, W = x_nchw.shape
  Cmid = w1_oihw.shape[0]
  Cout = w2_oihw.shape[0]
  HW = H * W
  PAD = 2 * W
  TILE = min(4096, HW)
  count = float(N * H * W)

  w1g = jnp.transpose(w1_oihw, (3, 0, 2, 1)).reshape(3, Cmid, 3 * Cin)
  w1g = w1g.astype(jnp.bfloat16)
  w2m = w2_oihw[:, :, 0, 0]

  NC = 2 if N % 2 == 0 else 1
  NPC = N // NC
  cparams = pltpu.CompilerParams(
      dimension_semantics=("parallel", "arbitrary"),
      vmem_limit_bytes=VMEM_LIMIT_BYTES)

  y, stats = pl.pallas_call(
      functools.partial(_conv_stats_kernel, w=W, hw=HW, pad=PAD, tile=TILE),
      out_shape=(jax.ShapeDtypeStruct((N, Cmid, HW), jnp.bfloat16),
                 jax.ShapeDtypeStruct((N, Cmid, 2), jnp.float32)),
      grid=(NC, NPC),
      in_specs=[
          pl.BlockSpec((None, Cin // 4, H, W),
                       functools.partial(
                           lambda q, c, i: (c * NPC + i, q, 0, 0), q))
          for q in range(4)
      ] + [
          pl.BlockSpec((3, Cmid, 3 * Cin), lambda c, i: (0, 0, 0)),
      ],
      out_specs=(pl.BlockSpec((None, Cmid, HW),
                              lambda c, i: (c * NPC + i, 0, 0)),
                 pl.BlockSpec((None, Cmid, 2),
                              lambda c, i: (c * NPC + i, 0, 0))),
      scratch_shapes=[pltpu.VMEM((Cin, PAD + HW + PAD), jnp.bfloat16),
                      pltpu.VMEM((3 * Cin, TILE), jnp.bfloat16)],
      compiler_params=cparams,
      cost_estimate=pl.CostEstimate(
          flops=int((2 * 9 * Cin * Cmid + 4 * Cmid) * N * HW),
          transcendentals=0,
          bytes_accessed=int(4 * N * Cin * HW + 2 * N * Cmid * HW)),
  )(x_nchw, x_nchw, x_nchw, x_nchw, w1g)

  s = jnp.sum(stats[:, :, 0], axis=0)
  s2 = jnp.sum(stats[:, :, 1], axis=0)
  mean = s / count
  var = s2 / count - mean * mean
  scale = gamma * jax.lax.rsqrt(var + EPS)
  shift = beta - mean * scale

  T2 = min(16384, HW)
  R2 = T2 // W
  NT2 = HW // T2
  cparams2 = pltpu.CompilerParams(
      dimension_semantics=("parallel", "arbitrary"),
      vmem_limit_bytes=VMEM_LIMIT_BYTES)
  out = pl.pallas_call(
      _apply_kernel,
      out_shape=jax.ShapeDtypeStruct((N, Cout, H, W), jnp.float32),
      grid=(NC, NPC * NT2),
      in_specs=[
          pl.BlockSpec((None, Cmid, T2),
                       lambda c, j: (c * NPC + j // NT2, 0, j % NT2)),
          pl.BlockSpec((Cmid, 1), lambda c, j: (0, 0)),
          pl.BlockSpec((Cmid, 1), lambda c, j: (0, 0)),
          pl.BlockSpec((Cout, Cmid), lambda c, j: (0, 0)),
          pl.BlockSpec((Cout, 1), lambda c, j: (0, 0)),
      ],
      out_specs=pl.BlockSpec(
          (None, Cout, R2, W),
          lambda c, j: (c * NPC + j // NT2, 0, j % NT2, 0)),
      compiler_params=cparams2,
      cost_estimate=pl.CostEstimate(
          flops=int((2 * Cmid * Cout + 2 * Cmid + 2 * Cout) * N * HW),
          transcendentals=int(Cout * N * HW),
          bytes_accessed=int(4 * (N * Cmid * HW + N * Cout * HW))),
  )(y, scale.reshape(Cmid, 1), shift.reshape(Cmid, 1), w2m,
    b2.reshape(Cout, 1))

  return out
